# Initial kernel scaffold; baseline (speedup 1.0000x reference)
import jax
import jax.numpy as jnp
from jax.experimental import pallas as pl


def kernel(inputs):
    raise NotImplementedError("write your pallas kernel here")


# speedup vs baseline: 1.1678x; 1.1678x over previous
import functools

import jax
import jax.numpy as jnp
from jax import lax
from jax.experimental import pallas as pl
from jax.experimental.pallas import tpu as pltpu

_CB = 128


def _stage(x, d, runlen):
    rows = x.shape[0]
    nblk = rows // (2 * d)
    y = x.reshape(nblk, 2, d, _CB)
    a, b = y[:, 0], y[:, 1]
    hi = jnp.maximum(a, b)
    lo = jnp.minimum(a, b)
    if runlen >= rows:
        first, second = hi, lo
    else:
        blk = lax.broadcasted_iota(jnp.int32, (nblk, 1, 1), 0)
        m = ((blk * (2 * d)) // runlen) % 2 == 0
        first = jnp.where(m, hi, lo)
        second = jnp.where(m, lo, hi)
    return jnp.stack([first, second], axis=1).reshape(rows, _CB)


def _topk_desc(x, k):
    rows = x.shape[0]
    run = 2
    while run <= k:
        d = run // 2
        while d >= 1:
            x = _stage(x, d, run)
            d //= 2
        run *= 2
    while rows > k:
        nblk = rows // (2 * k)
        y = x.reshape(nblk, 2, k, _CB)
        x = jnp.maximum(y[:, 0], y[:, 1]).reshape(-1, _CB)
        rows //= 2
        d = k // 2
        while d >= 1:
            x = _stage(x, d, k)
            d //= 2
    return x


def _kmax_body(x_ref, o_ref, *, k):
    o_ref[0] = _topk_desc(x_ref[0], k)


def kernel(inputs):
    B, S, C = inputs.shape
    k = 512
    body = functools.partial(_kmax_body, k=k)
    return pl.pallas_call(
        body,
        grid=(B, C // _CB),
        in_specs=[pl.BlockSpec((1, S, _CB), lambda b, c: (b, 0, c))],
        out_specs=pl.BlockSpec((1, k, _CB), lambda b, c: (b, 0, c)),
        out_shape=jax.ShapeDtypeStruct((B, k, C), jnp.float32),
    )(inputs)


# speedup vs baseline: 2.8971x; 2.4808x over previous
import numpy as np
import jax
import jax.numpy as jnp
from jax import lax
from jax.experimental import pallas as pl
from jax.experimental.pallas import tpu as pltpu
from jax.experimental.pallas import tpu_sc as plsc

_B = 4
_S = 4096
_C = 1024
_K = 512
_L = 16
_NC, _NS = 2, 16
_NW = _NC * _NS
_NG = _C // _L
_NTASK = _B * _NG
_TPW = _NTASK // _NW

_SIGN = np.int32(-2**31)


def _key_tf(kb):
    return kb ^ (lax.shift_right_arithmetic(kb, 31) & np.int32(0x7FFFFFFF))


def _digit(k, shift):
    if shift == 24:
        return lax.shift_right_logical(k ^ _SIGN, 24)
    return lax.shift_right_logical(k, shift) & np.int32(0xFF)


def _sc_body(inp, out, keys, hist, outa, outb):
    wid = lax.axis_index("s") * _NC + lax.axis_index("c")
    lane = lax.iota(jnp.int32, _L)
    zeros = jnp.zeros((_L,), jnp.int32)
    ones = jnp.ones((_L,), jnp.int32)

    def _zero(i, c):
        hist[i, :] = zeros
        return c
    lax.fori_loop(0, 256, _zero, 0)

    def _scan_hist(rank):
        def body(i, carry):
            cum, bsel, rnew = carry
            b_ = 255 - i
            h = hist[b_, :]
            hist[b_, :] = zeros
            cumh = cum + h
            cond = (cum < rank) & (cumh >= rank)
            bvec = lax.broadcast(b_, (_L,)).astype(jnp.int32)
            bsel = jnp.where(cond, bvec, bsel)
            rnew = jnp.where(cond, rank - cum, rnew)
            return (cumh, bsel, rnew)
        _, bsel, rnew = lax.fori_loop(0, 256, body, (zeros, zeros, zeros),
                                      unroll=4)
        return bsel, rnew

    def _task(t, carry):
        task = wid * _TPW + t
        b = task // _NG
        c0 = (task % _NG) * _L
        pltpu.sync_copy(inp.at[b, :, pl.ds(c0, _L)], keys)

        def p1(r, c):
            kb = plsc.bitcast(keys[r, :], jnp.int32)
            k = _key_tf(kb)
            keys[r, :] = plsc.bitcast(k, jnp.float32)
            plsc.addupdate_scatter(hist, [_digit(k, 24), lane], ones)
            return c
        lax.fori_loop(0, _S, p1, 0, unroll=8)
        b1, rank = _scan_hist(jnp.full((_L,), _K, jnp.int32))

        def classify(n_rows, bsel, shift, co, cc_bound, with_hist):
            def body(r, carry):
                co_, cc_ = carry
                kf = keys[r, :]
                k = plsc.bitcast(kf, jnp.int32)
                d = _digit(k, shift)
                if cc_bound is None:
                    valid = jnp.full((_L,), True)
                else:
                    valid = lax.broadcast(r, (_L,)) < cc_bound
                m_gt = valid & (d > bsel)
                m_eq = valid & (d == bsel)
                plsc.store_scatter(outa, [co_, lane], kf, mask=m_gt)
                plsc.store_scatter(keys, [cc_, lane], kf, mask=m_eq)
                if with_hist is not None:
                    plsc.addupdate_scatter(hist, [_digit(k, with_hist), lane],
                                           ones, mask=m_eq)
                co_ = co_ + jnp.where(m_gt, ones, zeros)
                cc_ = cc_ + jnp.where(m_eq, ones, zeros)
                return (co_, cc_)
            return lax.fori_loop(0, n_rows, body, (co, zeros),
                                 unroll=8 if cc_bound is None else None)

        co, cc = classify(_S, b1, 24, zeros, None, 16)
        b2, rank = _scan_hist(rank)
        co, cc = classify(jnp.max(cc), b2, 16, co, cc, 8)
        b3, rank = _scan_hist(rank)
        co, cc = classify(jnp.max(cc), b3, 8, co, cc, 0)
        b4, rank = _scan_hist(rank)
        co, cc = classify(jnp.max(cc), b4, 0, co, cc, None)

        tkey = ((b1 * 16777216) + (b2 * 65536) + (b3 * 256) + b4) ^ _SIGN
        tf = plsc.bitcast(tkey, jnp.float32)
        def fill(r, c):
            rv = lax.broadcast(r, (_L,))
            plsc.store_scatter(outa, [rv, lane], tf, mask=rv >= co)
            return c
        lax.fori_loop(jnp.min(co), _K, fill, 0)

        def sort_pass(src, dst, shift, last):
            def h_(r, c):
                k = plsc.bitcast(src[r, :], jnp.int32)
                plsc.addupdate_scatter(hist, [_digit(k, shift), lane], ones)
                return c
            lax.fori_loop(0, _K, h_, 0, unroll=8)

            def o_(i, cum):
                b_ = 255 - i
                h = hist[b_, :]
                hist[b_, :] = cum
                return cum + h
            lax.fori_loop(0, 256, o_, zeros, unroll=4)

            def r_(r, c):
                kf = src[r, :]
                k = plsc.bitcast(kf, jnp.int32)
                d = _digit(k, shift)
                pos = plsc.load_gather(hist, [d, lane])
                val = plsc.bitcast(_key_tf(k), jnp.float32) if last else kf
                plsc.store_scatter(dst, [pos, lane], val)
                plsc.store_scatter(hist, [d, lane], pos + ones)
                return c
            lax.fori_loop(0, _K, r_, 0, unroll=4)

            def z_(i, c):
                hist[i, :] = zeros
                return c
            lax.fori_loop(0, 256, z_, 0)

        sort_pass(outa, outb, 0, False)
        sort_pass(outb, outa, 8, False)
        sort_pass(outa, outb, 16, False)
        sort_pass(outb, outa, 24, True)

        pltpu.sync_copy(outa, out.at[b, :, pl.ds(c0, _L)])
        return carry

    lax.fori_loop(0, _TPW, _task, 0)


def kernel(inputs):
    mesh = plsc.VectorSubcoreMesh(core_axis_name="c", subcore_axis_name="s",
                                  num_cores=_NC, num_subcores=_NS)
    f = pl.kernel(
        _sc_body,
        out_type=jax.ShapeDtypeStruct((_B, _K, _C), jnp.float32),
        mesh=mesh,
        compiler_params=pltpu.CompilerParams(use_tc_tiling_on_sc=False,
                                             needs_layout_passes=False),
        scratch_types=[
            pltpu.VMEM((_S, _L), jnp.float32),
            pltpu.VMEM((256, _L), jnp.int32),
            pltpu.VMEM((_K, _L), jnp.float32),
            pltpu.VMEM((_K, _L), jnp.float32),
        ],
    )
    return f(inputs)


# speedup vs baseline: 5.1091x; 1.7635x over previous
import numpy as np
import jax
import jax.numpy as jnp
from jax import lax
from jax.experimental import pallas as pl
from jax.experimental.pallas import tpu as pltpu
from jax.experimental.pallas import tpu_sc as plsc

_B = 4
_S = 4096
_C = 1024
_K = 512
_L = 16
_NC, _NS = 2, 16
_NW = _NC * _NS
_NG = _C // _L
_NTASK = _B * _NG
_TPW = _NTASK // _NW

_SIGN = np.int32(-2**31)


def _key_tf(kb):
    return kb ^ (lax.shift_right_arithmetic(kb, 31) & np.int32(0x7FFFFFFF))


def _digit(k, shift):
    if shift == 24:
        return lax.shift_right_logical(k ^ _SIGN, 24)
    return lax.shift_right_logical(k, shift) & np.int32(0xFF)


def _sc_body(inp, out, buf, hist, offs, outb):
    wid = lax.axis_index("s") * _NC + lax.axis_index("c")
    lane = lax.iota(jnp.int32, _L)
    zeros = jnp.zeros((_L,), jnp.int32)
    ones = jnp.ones((_L,), jnp.int32)

    @plsc.parallel_loop(0, 256, unroll=4)
    def _zero(i):
        hist[i, :] = zeros

    def _scan_hist(rank):
        @plsc.parallel_loop(0, 256, unroll=4, carry=(zeros, zeros, zeros))
        def scan(i, carry):
            cum, bsel, rnew = carry
            b_ = 255 - i
            h = hist[b_, :]
            hist[b_, :] = zeros
            cumh = cum + h
            cond = (cum < rank) & (cumh >= rank)
            bvec = lax.broadcast(b_, (_L,)).astype(jnp.int32)
            bsel = jnp.where(cond, bvec, bsel)
            rnew = jnp.where(cond, rank - cum, rnew)
            return (cumh, bsel, rnew)
        _, bsel, rnew = scan
        return bsel, rnew

    def _task(t, carry):
        task = wid * _TPW + t
        b = task // _NG
        c0 = (task % _NG) * _L
        pltpu.sync_copy(inp.at[b, :, pl.ds(c0, _L)], buf.at[pl.ds(0, _S)])

        @plsc.parallel_loop(0, _S, unroll=8)
        def p1(r):
            kb = plsc.bitcast(buf[r, :], jnp.int32)
            plsc.addupdate_scatter(hist, [_digit(_key_tf(kb), 24), lane],
                                   ones)
        b1, rank = _scan_hist(jnp.full((_L,), _K, jnp.int32))

        def classify(n_rows, bsel, shift, co, cc_bound, with_hist, raw):
            def body(r, carry):
                co_, cc_ = carry
                kf = buf[r, :]
                k = plsc.bitcast(kf, jnp.int32)
                if raw:
                    k = _key_tf(k)
                    kf = plsc.bitcast(k, jnp.float32)
                d = _digit(k, shift)
                if cc_bound is None:
                    m_gt = d > bsel
                    m_eq = d == bsel
                else:
                    valid = lax.broadcast(r, (_L,)) < cc_bound
                    m_gt = valid & (d > bsel)
                    m_eq = valid & (d == bsel)
                idx = jnp.where(m_gt, co_ + _S, cc_)
                plsc.store_scatter(buf, [idx, lane], kf, mask=m_gt | m_eq)
                if with_hist is not None:
                    plsc.addupdate_scatter(hist, [_digit(k, with_hist), lane],
                                           ones, mask=m_eq)
                co_ = co_ + jnp.where(m_gt, ones, zeros)
                cc_ = cc_ + jnp.where(m_eq, ones, zeros)
                return (co_, cc_)
            if cc_bound is None:
                return plsc.parallel_loop(0, n_rows, unroll=8,
                                          carry=(co, zeros))(body)
            return lax.fori_loop(0, n_rows, body, (co, zeros))

        co, cc = classify(_S, b1, 24, zeros, None, 16, True)
        b2, rank = _scan_hist(rank)
        co, cc = classify(jnp.max(cc), b2, 16, co, cc, 8, False)
        b3, rank = _scan_hist(rank)
        co, cc = classify(jnp.max(cc), b3, 8, co, cc, 0, False)
        b4, rank = _scan_hist(rank)
        co, cc = classify(jnp.max(cc), b4, 0, co, cc, None, False)

        tkey = ((b1 * 16777216) + (b2 * 65536) + (b3 * 256) + b4) ^ _SIGN
        tf = plsc.bitcast(tkey, jnp.float32)

        @plsc.parallel_loop(jnp.min(co), _K)
        def fill(r):
            rv = lax.broadcast(r, (_L,))
            plsc.store_scatter(buf, [rv + _S, lane], tf, mask=rv >= co)

        def sort_pass(src, dst, shift, last):
            @plsc.parallel_loop(0, _K, unroll=8)
            def h_(r):
                if src == 0:
                    k = plsc.bitcast(buf[r + _S, :], jnp.int32)
                else:
                    k = plsc.bitcast(outb[r, :], jnp.int32)
                plsc.addupdate_scatter(hist, [_digit(k, shift), lane], ones)

            @plsc.parallel_loop(0, 256, unroll=4, carry=zeros)
            def o_(i, cum):
                b_ = 255 - i
                h = hist[b_, :]
                hist[b_, :] = zeros
                offs[b_, :] = cum
                return cum + h

            def r_(r, c):
                if src == 0:
                    kf = buf[r + _S, :]
                else:
                    kf = outb[r, :]
                k = plsc.bitcast(kf, jnp.int32)
                d = _digit(k, shift)
                pos = plsc.load_gather(offs, [d, lane])
                val = plsc.bitcast(_key_tf(k), jnp.float32) if last else kf
                if dst == 0:
                    plsc.store_scatter(buf, [pos + _S, lane], val)
                else:
                    plsc.store_scatter(outb, [pos, lane], val)
                plsc.store_scatter(offs, [d, lane], pos + ones)
                return c
            lax.fori_loop(0, _K, r_, 0, unroll=4)

        sort_pass(0, 1, 0, False)
        sort_pass(1, 0, 8, False)
        sort_pass(0, 1, 16, False)
        sort_pass(1, 0, 24, True)

        pltpu.sync_copy(buf.at[pl.ds(_S, _K)], out.at[b, :, pl.ds(c0, _L)])
        return carry

    lax.fori_loop(0, _TPW, _task, 0)


def kernel(inputs):
    mesh = plsc.VectorSubcoreMesh(core_axis_name="c", subcore_axis_name="s",
                                  num_cores=_NC, num_subcores=_NS)
    f = pl.kernel(
        _sc_body,
        out_type=jax.ShapeDtypeStruct((_B, _K, _C), jnp.float32),
        mesh=mesh,
        compiler_params=pltpu.CompilerParams(use_tc_tiling_on_sc=False,
                                             needs_layout_passes=False),
        scratch_types=[
            pltpu.VMEM((_S + _K, _L), jnp.float32),
            pltpu.VMEM((256, _L), jnp.int32),
            pltpu.VMEM((256, _L), jnp.int32),
            pltpu.VMEM((_K, _L), jnp.float32),
        ],
    )
    return f(inputs)


# speedup vs baseline: 5.6798x; 1.1117x over previous
import numpy as np
import jax
import jax.numpy as jnp
from jax import lax
from jax.experimental import pallas as pl
from jax.experimental.pallas import tpu as pltpu
from jax.experimental.pallas import tpu_sc as plsc

_B = 4
_S = 4096
_C = 1024
_K = 512
_L = 16
_NC, _NS = 2, 16
_NW = _NC * _NS
_NG = _C // _L
_NTASK = _B * _NG
_TPW = _NTASK // _NW

_SIGN = np.int32(-2**31)


def _key_tf(kb):
    return kb ^ (lax.shift_right_arithmetic(kb, 31) & np.int32(0x7FFFFFFF))


def _digit(k, shift):
    if shift == 24:
        return lax.shift_right_logical(k ^ _SIGN, 24)
    return lax.shift_right_logical(k, shift) & np.int32(0xFF)


def _sc_body(inp, out, buf, hist, offs, outb):
    wid = lax.axis_index("s") * _NC + lax.axis_index("c")
    lane = lax.iota(jnp.int32, _L)
    zeros = jnp.zeros((_L,), jnp.int32)
    ones = jnp.ones((_L,), jnp.int32)

    @plsc.parallel_loop(0, 256, unroll=4)
    def _zero(i):
        hist[i, :] = zeros

    def _scan_hist(rank):
        @plsc.parallel_loop(0, 256, unroll=4, carry=(zeros, zeros, zeros))
        def scan(i, carry):
            cum, bsel, rnew = carry
            b_ = 255 - i
            h = hist[b_, :]
            hist[b_, :] = zeros
            cumh = cum + h
            cond = (cum < rank) & (cumh >= rank)
            bvec = lax.broadcast(b_, (_L,)).astype(jnp.int32)
            bsel = jnp.where(cond, bvec, bsel)
            rnew = jnp.where(cond, rank - cum, rnew)
            return (cumh, bsel, rnew)
        _, bsel, rnew = scan
        return bsel, rnew

    def _task(t, carry):
        task = wid * _TPW + t
        b = task // _NG
        c0 = (task % _NG) * _L
        pltpu.sync_copy(inp.at[b, :, pl.ds(c0, _L)], buf.at[pl.ds(0, _S)])

        @plsc.parallel_loop(0, _S, unroll=8)
        def p1(r):
            kb = plsc.bitcast(buf[r, :], jnp.int32)
            plsc.addupdate_scatter(hist, [_digit(_key_tf(kb), 24), lane],
                                   ones)
        b1, rank = _scan_hist(jnp.full((_L,), _K, jnp.int32))

        def classify(n_rows, bsel, shift, co, cc_bound, with_hist, raw):
            def body(r, carry):
                co_, cc_ = carry
                kf = buf[r, :]
                k = plsc.bitcast(kf, jnp.int32)
                if raw:
                    k = _key_tf(k)
                    kf = plsc.bitcast(k, jnp.float32)
                d = _digit(k, shift)
                if cc_bound is None:
                    m_gt = d > bsel
                    m_eq = d == bsel
                else:
                    valid = lax.broadcast(r, (_L,)) < cc_bound
                    m_gt = valid & (d > bsel)
                    m_eq = valid & (d == bsel)
                idx = jnp.where(m_gt, co_ + _S, cc_)
                plsc.store_scatter(buf, [idx, lane], kf, mask=m_gt | m_eq)
                if with_hist is not None:
                    plsc.addupdate_scatter(hist, [_digit(k, with_hist), lane],
                                           ones, mask=m_eq)
                co_ = co_ + jnp.where(m_gt, ones, zeros)
                cc_ = cc_ + jnp.where(m_eq, ones, zeros)
                return (co_, cc_)
            return plsc.parallel_loop(0, n_rows,
                                      unroll=8 if cc_bound is None else 4,
                                      carry=(co, zeros))(body)

        co, cc = classify(_S, b1, 24, zeros, None, 16, True)
        b2, rank = _scan_hist(rank)
        co, cc = classify(jnp.max(cc), b2, 16, co, cc, 8, False)
        b3, rank = _scan_hist(rank)
        co, cc = classify(jnp.max(cc), b3, 8, co, cc, 0, False)
        b4, rank = _scan_hist(rank)
        co, cc = classify(jnp.max(cc), b4, 0, co, cc, None, False)

        tkey = ((b1 * 16777216) + (b2 * 65536) + (b3 * 256) + b4) ^ _SIGN
        tf = plsc.bitcast(tkey, jnp.float32)

        @plsc.parallel_loop(jnp.min(co), _K)
        def fill(r):
            rv = lax.broadcast(r, (_L,))
            plsc.store_scatter(buf, [rv + _S, lane], tf, mask=rv >= co)

        def sort_pass(src, dst, shift, last):
            @plsc.parallel_loop(0, _K, unroll=8)
            def h_(r):
                if src == 0:
                    k = plsc.bitcast(buf[r + _S, :], jnp.int32)
                else:
                    k = plsc.bitcast(outb[r, :], jnp.int32)
                plsc.addupdate_scatter(hist, [_digit(k, shift), lane], ones)

            @plsc.parallel_loop(0, 256, unroll=4, carry=zeros)
            def o_(i, cum):
                b_ = 255 - i
                h = hist[b_, :]
                hist[b_, :] = zeros
                offs[b_, :] = cum
                return cum + h

            def r_(r, c):
                if src == 0:
                    kf = buf[r + _S, :]
                else:
                    kf = outb[r, :]
                k = plsc.bitcast(kf, jnp.int32)
                d = _digit(k, shift)
                pos = plsc.load_gather(offs, [d, lane])
                val = plsc.bitcast(_key_tf(k), jnp.float32) if last else kf
                if dst == 0:
                    plsc.store_scatter(buf, [pos + _S, lane], val)
                else:
                    plsc.store_scatter(outb, [pos, lane], val)
                plsc.store_scatter(offs, [d, lane], pos + ones)
                return c
            lax.fori_loop(0, _K, r_, 0, unroll=4)

        sort_pass(0, 1, 0, False)
        sort_pass(1, 0, 8, False)
        sort_pass(0, 1, 16, False)
        sort_pass(1, 0, 24, True)

        pltpu.sync_copy(buf.at[pl.ds(_S, _K)], out.at[b, :, pl.ds(c0, _L)])
        return carry

    lax.fori_loop(0, _TPW, _task, 0)


def kernel(inputs):
    mesh = plsc.VectorSubcoreMesh(core_axis_name="c", subcore_axis_name="s",
                                  num_cores=_NC, num_subcores=_NS)
    f = pl.kernel(
        _sc_body,
        out_type=jax.ShapeDtypeStruct((_B, _K, _C), jnp.float32),
        mesh=mesh,
        compiler_params=pltpu.CompilerParams(use_tc_tiling_on_sc=False,
                                             needs_layout_passes=False),
        scratch_types=[
            pltpu.VMEM((_S + _K, _L), jnp.float32),
            pltpu.VMEM((256, _L), jnp.int32),
            pltpu.VMEM((256, _L), jnp.int32),
            pltpu.VMEM((_K, _L), jnp.float32),
        ],
    )
    return f(inputs)


# speedup vs baseline: 7.4708x; 1.3153x over previous
import numpy as np
import jax
import jax.numpy as jnp
from jax import lax
from jax.experimental import pallas as pl
from jax.experimental.pallas import tpu as pltpu
from jax.experimental.pallas import tpu_sc as plsc

_B = 4
_S = 4096
_C = 1024
_K = 512
_L = 16
_NC, _NS = 2, 16
_NW = _NC * _NS
_NG = _C // _L
_NTASK = _B * _NG
_TPW = _NTASK // _NW

_SIGN = np.int32(-2**31)


def _key_tf(kb):
    return kb ^ (lax.shift_right_arithmetic(kb, 31) & np.int32(0x7FFFFFFF))


def _digit(k, shift):
    if shift == 24:
        return lax.shift_right_logical(k ^ _SIGN, 24)
    return lax.shift_right_logical(k, shift) & np.int32(0xFF)


def _sc_body(inp, out, buf, hist, offs, outb):
    wid = lax.axis_index("s") * _NC + lax.axis_index("c")
    lane = lax.iota(jnp.int32, _L)
    zeros = jnp.zeros((_L,), jnp.int32)
    ones = jnp.ones((_L,), jnp.int32)

    @plsc.parallel_loop(0, 256, unroll=4)
    def _zero(i):
        hist[i, :] = zeros

    def _scan_hist(rank):
        @plsc.parallel_loop(0, 256, unroll=4, carry=(zeros, zeros, zeros))
        def scan(i, carry):
            cum, bsel, rnew = carry
            b_ = 255 - i
            h = hist[b_, :]
            hist[b_, :] = zeros
            cumh = cum + h
            cond = (cum < rank) & (cumh >= rank)
            bvec = lax.broadcast(b_, (_L,)).astype(jnp.int32)
            bsel = jnp.where(cond, bvec, bsel)
            rnew = jnp.where(cond, rank - cum, rnew)
            return (cumh, bsel, rnew)
        _, bsel, rnew = scan
        return bsel, rnew

    def _task(t, carry):
        task = wid * _TPW + t
        b = task // _NG
        c0 = (task % _NG) * _L
        pltpu.sync_copy(inp.at[b, :, pl.ds(c0, _L)], buf.at[pl.ds(0, _S)])

        @plsc.parallel_loop(0, _S, unroll=8)
        def p1(r):
            kb = plsc.bitcast(buf[r, :], jnp.int32)
            plsc.addupdate_scatter(hist, [_digit(_key_tf(kb), 24), lane],
                                   ones)
        b1, rank = _scan_hist(jnp.full((_L,), _K, jnp.int32))

        def classify(n_rows, bsel, shift, co, cc_bound, with_hist, raw):
            def body(r, carry):
                co_, cc_ = carry
                kf = buf[r, :]
                k = plsc.bitcast(kf, jnp.int32)
                if raw:
                    k = _key_tf(k)
                    kf = plsc.bitcast(k, jnp.float32)
                d = _digit(k, shift)
                if cc_bound is None:
                    m_gt = d > bsel
                    m_eq = d == bsel
                else:
                    valid = lax.broadcast(r, (_L,)) < cc_bound
                    m_gt = valid & (d > bsel)
                    m_eq = valid & (d == bsel)
                idx = jnp.where(m_gt, co_ + _S, cc_)
                plsc.store_scatter(buf, [idx, lane], kf, mask=m_gt | m_eq)
                if with_hist is not None:
                    plsc.addupdate_scatter(hist, [_digit(k, with_hist), lane],
                                           ones, mask=m_eq)
                co_ = co_ + jnp.where(m_gt, ones, zeros)
                cc_ = cc_ + jnp.where(m_eq, ones, zeros)
                return (co_, cc_)
            return plsc.parallel_loop(0, n_rows,
                                      unroll=8 if cc_bound is None else 4,
                                      carry=(co, zeros))(body)

        co, cc = classify(_S, b1, 24, zeros, None, 16, True)
        b2, rank = _scan_hist(rank)
        co, cc = classify(jnp.max(cc), b2, 16, co, cc, 8, False)
        b3, rank = _scan_hist(rank)
        co, cc = classify(jnp.max(cc), b3, 8, co, cc, 0, False)
        b4, rank = _scan_hist(rank)
        co, cc = classify(jnp.max(cc), b4, 0, co, cc, None, False)

        tkey = ((b1 * 16777216) + (b2 * 65536) + (b3 * 256) + b4) ^ _SIGN
        tf = plsc.bitcast(tkey, jnp.float32)

        @plsc.parallel_loop(jnp.min(co), _K)
        def fill(r):
            rv = lax.broadcast(r, (_L,))
            plsc.store_scatter(buf, [rv + _S, lane], tf, mask=rv >= co)

        def sort_pass(src, dst, shift, last):
            @plsc.parallel_loop(0, _K, unroll=8)
            def h_(r):
                if src == 0:
                    k = plsc.bitcast(buf[r + _S, :], jnp.int32)
                else:
                    k = plsc.bitcast(outb[r, :], jnp.int32)
                plsc.addupdate_scatter(hist, [_digit(k, shift), lane], ones)

            @plsc.parallel_loop(0, 256, unroll=4, carry=zeros)
            def o_(i, cum):
                b_ = 255 - i
                h = hist[b_, :]
                hist[b_, :] = zeros
                offs[b_, :] = cum
                return cum + h

            def r_(g, c):
                r0 = g * 4
                kf, k, d, pos = [], [], [], []
                for i in range(4):
                    if src == 0:
                        kfi = buf[r0 + i + _S, :]
                    else:
                        kfi = outb[r0 + i, :]
                    ki = plsc.bitcast(kfi, jnp.int32)
                    kf.append(kfi)
                    k.append(ki)
                    d.append(_digit(ki, shift))
                    pos.append(plsc.load_gather(offs, [d[i], lane]))
                for i in range(1, 4):
                    adj = zeros
                    for j in range(i):
                        adj = adj + jnp.where(d[i] == d[j], ones, zeros)
                    pos[i] = pos[i] + adj
                for i in range(4):
                    val = (plsc.bitcast(_key_tf(k[i]), jnp.float32)
                           if last else kf[i])
                    if dst == 0:
                        plsc.store_scatter(buf, [pos[i] + _S, lane], val)
                    else:
                        plsc.store_scatter(outb, [pos[i], lane], val)
                for i in range(4):
                    plsc.store_scatter(offs, [d[i], lane], pos[i] + ones)
                return c
            lax.fori_loop(0, _K // 4, r_, 0, unroll=2)

        sort_pass(0, 1, 0, False)
        sort_pass(1, 0, 8, False)
        sort_pass(0, 1, 16, False)
        sort_pass(1, 0, 24, True)

        pltpu.sync_copy(buf.at[pl.ds(_S, _K)], out.at[b, :, pl.ds(c0, _L)])
        return carry

    lax.fori_loop(0, _TPW, _task, 0)


def kernel(inputs):
    mesh = plsc.VectorSubcoreMesh(core_axis_name="c", subcore_axis_name="s",
                                  num_cores=_NC, num_subcores=_NS)
    f = pl.kernel(
        _sc_body,
        out_type=jax.ShapeDtypeStruct((_B, _K, _C), jnp.float32),
        mesh=mesh,
        compiler_params=pltpu.CompilerParams(use_tc_tiling_on_sc=False,
                                             needs_layout_passes=False),
        scratch_types=[
            pltpu.VMEM((_S + _K, _L), jnp.float32),
            pltpu.VMEM((256, _L), jnp.int32),
            pltpu.VMEM((256, _L), jnp.int32),
            pltpu.VMEM((_K, _L), jnp.float32),
        ],
    )
    return f(inputs)


# speedup vs baseline: 8.0840x; 1.0821x over previous
import numpy as np
import jax
import jax.numpy as jnp
from jax import lax
from jax.experimental import pallas as pl
from jax.experimental.pallas import tpu as pltpu
from jax.experimental.pallas import tpu_sc as plsc

_B = 4
_S = 4096
_C = 1024
_K = 512
_L = 16
_NC, _NS = 2, 16
_NW = _NC * _NS
_NG = _C // _L
_NTASK = _B * _NG
_TPW = _NTASK // _NW

_SIGN = np.int32(-2**31)


def _key_tf(kb):
    return kb ^ (lax.shift_right_arithmetic(kb, 31) & np.int32(0x7FFFFFFF))


def _digit(k, shift):
    if shift == 24:
        return lax.shift_right_logical(k ^ _SIGN, 24)
    return lax.shift_right_logical(k, shift) & np.int32(0xFF)


def _sc_body(inp, out, buf, hist, offs, outb, sem_in):
    wid = lax.axis_index("s") * _NC + lax.axis_index("c")
    lane = lax.iota(jnp.int32, _L)
    zeros = jnp.zeros((_L,), jnp.int32)
    ones = jnp.ones((_L,), jnp.int32)

    def _in_slice(task):
        b = task // _NG
        c0 = (task % _NG) * _L
        return inp.at[b, :, pl.ds(c0, _L)]

    pltpu.async_copy(_in_slice(wid * _TPW), buf.at[pl.ds(0, _S)], sem_in)

    @plsc.parallel_loop(0, 256, unroll=4)
    def _zero(i):
        hist[i, :] = zeros

    def _scan_hist(rank):
        @plsc.parallel_loop(0, 256, unroll=4, carry=(zeros, zeros, zeros))
        def scan(i, carry):
            cum, bsel, rnew = carry
            b_ = 255 - i
            h = hist[b_, :]
            hist[b_, :] = zeros
            cumh = cum + h
            cond = (cum < rank) & (cumh >= rank)
            bvec = lax.broadcast(b_, (_L,)).astype(jnp.int32)
            bsel = jnp.where(cond, bvec, bsel)
            rnew = jnp.where(cond, rank - cum, rnew)
            return (cumh, bsel, rnew)
        _, bsel, rnew = scan
        return bsel, rnew

    def _task(t, carry):
        task = wid * _TPW + t
        b = task // _NG
        c0 = (task % _NG) * _L
        pltpu.make_async_copy(_in_slice(task), buf.at[pl.ds(0, _S)],
                              sem_in).wait()

        @plsc.parallel_loop(0, _S, unroll=8)
        def p1(r):
            kb = plsc.bitcast(buf[r, :], jnp.int32)
            plsc.addupdate_scatter(hist, [_digit(_key_tf(kb), 24), lane],
                                   ones)
        b1, rank = _scan_hist(jnp.full((_L,), _K, jnp.int32))

        def classify(n_rows, bsel, shift, co, cc_bound, with_hist, raw):
            def body(r, carry):
                co_, cc_ = carry
                kf = buf[r, :]
                k = plsc.bitcast(kf, jnp.int32)
                if raw:
                    k = _key_tf(k)
                    kf = plsc.bitcast(k, jnp.float32)
                d = _digit(k, shift)
                if cc_bound is None:
                    m_gt = d > bsel
                    m_eq = d == bsel
                else:
                    valid = lax.broadcast(r, (_L,)) < cc_bound
                    m_gt = valid & (d > bsel)
                    m_eq = valid & (d == bsel)
                idx = jnp.where(m_gt, co_ + _S, cc_)
                plsc.store_scatter(buf, [idx, lane], kf, mask=m_gt | m_eq)
                if with_hist is not None:
                    plsc.addupdate_scatter(hist, [_digit(k, with_hist), lane],
                                           ones, mask=m_eq)
                co_ = co_ + jnp.where(m_gt, ones, zeros)
                cc_ = cc_ + jnp.where(m_eq, ones, zeros)
                return (co_, cc_)
            return plsc.parallel_loop(0, n_rows,
                                      unroll=8 if cc_bound is None else 4,
                                      carry=(co, zeros))(body)

        co, cc = classify(_S, b1, 24, zeros, None, 16, True)
        b2, rank = _scan_hist(rank)
        co, cc = classify(jnp.max(cc), b2, 16, co, cc, 8, False)
        b3, rank = _scan_hist(rank)
        co, cc = classify(jnp.max(cc), b3, 8, co, cc, 0, False)
        b4, rank = _scan_hist(rank)
        co, cc = classify(jnp.max(cc), b4, 0, co, cc, None, False)

        tkey = ((b1 * 16777216) + (b2 * 65536) + (b3 * 256) + b4) ^ _SIGN
        tf = plsc.bitcast(tkey, jnp.float32)

        @plsc.parallel_loop(jnp.min(co), _K)
        def fill(r):
            rv = lax.broadcast(r, (_L,))
            plsc.store_scatter(buf, [rv + _S, lane], tf, mask=rv >= co)

        nxt = jnp.minimum(t + 1, _TPW - 1) + wid * _TPW
        pltpu.async_copy(_in_slice(nxt), buf.at[pl.ds(0, _S)], sem_in)

        def sort_pass(src, dst, shift, last):
            @plsc.parallel_loop(0, _K, unroll=8)
            def h_(r):
                if src == 0:
                    k = plsc.bitcast(buf[r + _S, :], jnp.int32)
                else:
                    k = plsc.bitcast(outb[r, :], jnp.int32)
                plsc.addupdate_scatter(hist, [_digit(k, shift), lane], ones)

            @plsc.parallel_loop(0, 256, unroll=4, carry=zeros)
            def o_(i, cum):
                b_ = 255 - i
                h = hist[b_, :]
                hist[b_, :] = zeros
                offs[b_, :] = cum
                return cum + h

            def r_(g, c):
                r0 = g * 4
                kf, k, d, pos = [], [], [], []
                for i in range(4):
                    if src == 0:
                        kfi = buf[r0 + i + _S, :]
                    else:
                        kfi = outb[r0 + i, :]
                    ki = plsc.bitcast(kfi, jnp.int32)
                    kf.append(kfi)
                    k.append(ki)
                    d.append(_digit(ki, shift))
                    pos.append(plsc.load_gather(offs, [d[i], lane]))
                for i in range(1, 4):
                    adj = zeros
                    for j in range(i):
                        adj = adj + jnp.where(d[i] == d[j], ones, zeros)
                    pos[i] = pos[i] + adj
                for i in range(4):
                    val = (plsc.bitcast(_key_tf(k[i]), jnp.float32)
                           if last else kf[i])
                    if dst == 0:
                        plsc.store_scatter(buf, [pos[i] + _S, lane], val)
                    else:
                        plsc.store_scatter(outb, [pos[i], lane], val)
                for i in range(4):
                    plsc.store_scatter(offs, [d[i], lane], pos[i] + ones)
                return c
            lax.fori_loop(0, _K // 4, r_, 0, unroll=2)

        sort_pass(0, 1, 0, False)
        sort_pass(1, 0, 8, False)
        sort_pass(0, 1, 16, False)
        sort_pass(1, 0, 24, True)

        pltpu.sync_copy(buf.at[pl.ds(_S, _K)], out.at[b, :, pl.ds(c0, _L)])
        return carry

    lax.fori_loop(0, _TPW, _task, 0)
    pltpu.make_async_copy(_in_slice(wid * _TPW + _TPW - 1),
                          buf.at[pl.ds(0, _S)], sem_in).wait()


def kernel(inputs):
    mesh = plsc.VectorSubcoreMesh(core_axis_name="c", subcore_axis_name="s",
                                  num_cores=_NC, num_subcores=_NS)
    f = pl.kernel(
        _sc_body,
        out_type=jax.ShapeDtypeStruct((_B, _K, _C), jnp.float32),
        mesh=mesh,
        compiler_params=pltpu.CompilerParams(use_tc_tiling_on_sc=False,
                                             needs_layout_passes=False),
        scratch_types=[
            pltpu.VMEM((_S + _K, _L), jnp.float32),
            pltpu.VMEM((256, _L), jnp.int32),
            pltpu.VMEM((256, _L), jnp.int32),
            pltpu.VMEM((_K, _L), jnp.float32),
            pltpu.SemaphoreType.DMA,
        ],
    )
    return f(inputs)


# speedup vs baseline: 9.0790x; 1.1231x over previous
import numpy as np
import jax
import jax.numpy as jnp
from jax import lax
from jax.experimental import pallas as pl
from jax.experimental.pallas import tpu as pltpu
from jax.experimental.pallas import tpu_sc as plsc

_B = 4
_S = 4096
_C = 1024
_K = 512
_L = 16
_NC, _NS = 2, 16
_NW = _NC * _NS
_NG = _C // _L
_NTASK = _B * _NG
_TPW = _NTASK // _NW

_SIGN = np.int32(-2**31)


def _key_tf(kb):
    return kb ^ (lax.shift_right_arithmetic(kb, 31) & np.int32(0x7FFFFFFF))


def _digit(k, shift):
    if shift == 24:
        return lax.shift_right_logical(k ^ _SIGN, 24)
    return lax.shift_right_logical(k, shift) & np.int32(0xFF)


def _sc_body(inp, out, buf, hist, offs, outb, sem_in):
    wid = lax.axis_index("s") * _NC + lax.axis_index("c")
    lane = lax.iota(jnp.int32, _L)
    zeros = jnp.zeros((_L,), jnp.int32)
    ones = jnp.ones((_L,), jnp.int32)

    def _in_slice(task):
        b = task // _NG
        c0 = (task % _NG) * _L
        return inp.at[b, :, pl.ds(c0, _L)]

    pltpu.async_copy(_in_slice(wid * _TPW), buf.at[pl.ds(0, _S)], sem_in)

    @plsc.parallel_loop(0, 256, unroll=4)
    def _zero(i):
        hist[i, :] = zeros

    def _scan_hist(rank):
        @plsc.parallel_loop(0, 256, unroll=4, carry=(zeros, zeros, zeros))
        def scan(i, carry):
            cum, bsel, rnew = carry
            b_ = 255 - i
            h = hist[b_, :]
            hist[b_, :] = zeros
            cumh = cum + h
            cond = (cum < rank) & (cumh >= rank)
            bvec = lax.broadcast(b_, (_L,)).astype(jnp.int32)
            bsel = jnp.where(cond, bvec, bsel)
            rnew = jnp.where(cond, rank - cum, rnew)
            return (cumh, bsel, rnew)
        _, bsel, rnew = scan
        return bsel, rnew

    def _task(t, carry):
        task = wid * _TPW + t
        b = task // _NG
        c0 = (task % _NG) * _L
        pltpu.make_async_copy(_in_slice(task), buf.at[pl.ds(0, _S)],
                              sem_in).wait()

        @plsc.parallel_loop(0, _S, unroll=8)
        def p1(r):
            kb = plsc.bitcast(buf[r, :], jnp.int32)
            plsc.addupdate_scatter(hist, [_digit(_key_tf(kb), 24), lane],
                                   ones)
        b1, rank = _scan_hist(jnp.full((_L,), _K, jnp.int32))

        def classify(n_rows, bsel, shift, co, cc_bound, raw):
            def body(r, carry):
                co_, cc_ = carry
                kf = buf[r, :]
                k = plsc.bitcast(kf, jnp.int32)
                if raw:
                    k = _key_tf(k)
                    kf = plsc.bitcast(k, jnp.float32)
                d = _digit(k, shift)
                if cc_bound is None:
                    m_gt = d > bsel
                    m_eq = d == bsel
                else:
                    valid = lax.broadcast(r, (_L,)) < cc_bound
                    m_gt = valid & (d > bsel)
                    m_eq = valid & (d == bsel)
                idx = jnp.where(m_gt, co_ + _S, cc_)
                plsc.store_scatter(buf, [idx, lane], kf, mask=m_gt | m_eq)
                co_ = co_ + jnp.where(m_gt, ones, zeros)
                cc_ = cc_ + jnp.where(m_eq, ones, zeros)
                return (co_, cc_)
            return plsc.parallel_loop(0, n_rows,
                                      unroll=8 if cc_bound is None else 4,
                                      carry=(co, zeros))(body)

        def cand_hist(cc, shift):
            @plsc.parallel_loop(0, jnp.max(cc), unroll=4)
            def h(r):
                k = plsc.bitcast(buf[r, :], jnp.int32)
                plsc.addupdate_scatter(hist, [_digit(k, shift), lane], ones,
                                       mask=lax.broadcast(r, (_L,)) < cc)

        co, cc = classify(_S, b1, 24, zeros, None, True)
        cand_hist(cc, 16)
        b2, rank = _scan_hist(rank)
        co, cc = classify(jnp.max(cc), b2, 16, co, cc, False)
        cand_hist(cc, 8)
        b3, rank = _scan_hist(rank)
        co, cc = classify(jnp.max(cc), b3, 8, co, cc, False)
        cand_hist(cc, 0)
        b4, rank = _scan_hist(rank)
        co, cc = classify(jnp.max(cc), b4, 0, co, cc, False)

        tkey = ((b1 * 16777216) + (b2 * 65536) + (b3 * 256) + b4) ^ _SIGN
        tf = plsc.bitcast(tkey, jnp.float32)

        @plsc.parallel_loop(jnp.min(co), _K)
        def fill(r):
            rv = lax.broadcast(r, (_L,))
            plsc.store_scatter(buf, [rv + _S, lane], tf, mask=rv >= co)

        nxt = jnp.minimum(t + 1, _TPW - 1) + wid * _TPW
        pltpu.async_copy(_in_slice(nxt), buf.at[pl.ds(0, _S)], sem_in)

        def sort_pass(src, dst, shift, last):
            @plsc.parallel_loop(0, _K, unroll=8)
            def h_(r):
                if src == 0:
                    k = plsc.bitcast(buf[r + _S, :], jnp.int32)
                else:
                    k = plsc.bitcast(outb[r, :], jnp.int32)
                plsc.addupdate_scatter(hist, [_digit(k, shift), lane], ones)

            @plsc.parallel_loop(0, 256, unroll=4, carry=zeros)
            def o_(i, cum):
                b_ = 255 - i
                h = hist[b_, :]
                hist[b_, :] = zeros
                offs[b_, :] = cum
                return cum + h

            def r_(g, c):
                r0 = g * 8
                kf, k, d, pos = [], [], [], []
                for i in range(8):
                    if src == 0:
                        kfi = buf[r0 + i + _S, :]
                    else:
                        kfi = outb[r0 + i, :]
                    ki = plsc.bitcast(kfi, jnp.int32)
                    kf.append(kfi)
                    k.append(ki)
                    d.append(_digit(ki, shift))
                    pos.append(plsc.load_gather(offs, [d[i], lane]))
                for i in range(1, 8):
                    adj = zeros
                    for j in range(i):
                        adj = adj + jnp.where(d[i] == d[j], ones, zeros)
                    pos[i] = pos[i] + adj
                for i in range(8):
                    val = (plsc.bitcast(_key_tf(k[i]), jnp.float32)
                           if last else kf[i])
                    if dst == 0:
                        plsc.store_scatter(buf, [pos[i] + _S, lane], val)
                    else:
                        plsc.store_scatter(outb, [pos[i], lane], val)
                for i in range(8):
                    plsc.store_scatter(offs, [d[i], lane], pos[i] + ones)
                return c
            lax.fori_loop(0, _K // 8, r_, 0)

        sort_pass(0, 1, 0, False)
        sort_pass(1, 0, 8, False)
        sort_pass(0, 1, 16, False)
        sort_pass(1, 0, 24, True)

        pltpu.sync_copy(buf.at[pl.ds(_S, _K)], out.at[b, :, pl.ds(c0, _L)])
        return carry

    lax.fori_loop(0, _TPW, _task, 0)
    pltpu.make_async_copy(_in_slice(wid * _TPW + _TPW - 1),
                          buf.at[pl.ds(0, _S)], sem_in).wait()


def kernel(inputs):
    mesh = plsc.VectorSubcoreMesh(core_axis_name="c", subcore_axis_name="s",
                                  num_cores=_NC, num_subcores=_NS)
    f = pl.kernel(
        _sc_body,
        out_type=jax.ShapeDtypeStruct((_B, _K, _C), jnp.float32),
        mesh=mesh,
        compiler_params=pltpu.CompilerParams(use_tc_tiling_on_sc=False,
                                             needs_layout_passes=False),
        scratch_types=[
            pltpu.VMEM((_S + _K, _L), jnp.float32),
            pltpu.VMEM((256, _L), jnp.int32),
            pltpu.VMEM((256, _L), jnp.int32),
            pltpu.VMEM((_K, _L), jnp.float32),
            pltpu.SemaphoreType.DMA,
        ],
    )
    return f(inputs)
